# baseline (speedup 1.0000x reference)
import jax
import jax.numpy as jnp
from jax import lax
from jax.experimental import pallas as pl
from jax.experimental.pallas import tpu as pltpu
from jax.experimental.pallas import tpu_sc as plsc

N_NODES = 10000
N_EDGES = 320000
CH = 128

NC = 2
NS = 16
NW = NC * NS
E_PER_W = N_EDGES // NW
CHUNK = 40
N_CHUNKS = E_PER_W // CHUNK
MBLK = 50
NMBLK = N_CHUNKS // MBLK
ZROWS = 40
NZBLOCKS = N_NODES // ZROWS
NLANE = 16
WPAD = ((MBLK * CHUNK + NLANE + 127) // 128) * 128


def _sc_body(x_hbm, srcm_hbm, dstm_hbm, wm_hbm, out_hbm,
             srcm, dstm, wflat, gbuf_a, gbuf_b, sbuf_a, sbuf_b, acc,
             gsem_a, gsem_b, ssem_a, ssem_b):
    c = lax.axis_index("c")
    s = lax.axis_index("s")
    wid = c * NS + s
    gbufs = (gbuf_a, gbuf_b)
    sbufs = (sbuf_a, sbuf_b)
    gsems = (gsem_a, gsem_b)
    ssems = (ssem_a, ssem_b)

    zero16 = jnp.zeros((NLANE,), jnp.float32)

    def zrow(i, carry):
        for k in range(CH // NLANE):
            gbuf_a[i, pl.ds(k * NLANE, NLANE)] = zero16
        return carry

    lax.fori_loop(0, ZROWS, zrow, 0)
    for k in range((NZBLOCKS + NS - 1) // NS):
        b = s + NS * k
        r0 = pl.multiple_of(b * ZROWS, 8)
        if (NS * k) + NS <= NZBLOCKS:
            pltpu.sync_copy(gbuf_a, acc.at[pl.ds(r0, ZROWS)])
        else:
            @pl.when(b < NZBLOCKS)
            def _():
                pltpu.sync_copy(gbuf_a, acc.at[pl.ds(r0, ZROWS)])
    plsc.subcore_barrier()

    def do_chunk(i, b, first, refill):
        pltpu.make_async_copy(x_hbm.at[srcm.at[i]], gbufs[b], gsems[b]).wait()
        if not first:
            pltpu.make_async_copy(sbufs[b], acc.at[dstm.at[i]], ssems[b]).wait()
        off0 = i * CHUNK

        def scale(e, carry2):
            w16 = wflat[0, pl.ds(off0 + e, NLANE)]
            wv = jnp.full((NLANE,), w16[0])
            for k in range(CH // NLANE):
                sl = pl.ds(k * NLANE, NLANE)
                sbufs[b][e, sl] = gbufs[b][e, sl] * wv
            return carry2

        lax.fori_loop(0, CHUNK, scale, 0)
        pltpu.async_copy(sbufs[b], acc.at[dstm.at[i]], ssems[b], add=True)
        if refill:
            @pl.when(i + 2 < MBLK)
            def _():
                pltpu.async_copy(x_hbm.at[srcm.at[i + 2]], gbufs[b], gsems[b])

    for mb in range(NMBLK):
        pltpu.sync_copy(srcm_hbm.at[wid, mb], srcm)
        pltpu.sync_copy(dstm_hbm.at[wid, mb], dstm)
        pltpu.sync_copy(wm_hbm.at[wid, mb], wflat)
        pltpu.async_copy(x_hbm.at[srcm.at[0]], gbuf_a, gsem_a)
        pltpu.async_copy(x_hbm.at[srcm.at[1]], gbuf_b, gsem_b)
        do_chunk(0, 0, first=True, refill=True)
        do_chunk(1, 1, first=True, refill=True)

        def pair_body(i2, carry):
            for b in range(2):
                do_chunk(i2 * 2 + b, b, first=False, refill=True)
            return carry

        lax.fori_loop(1, MBLK // 2, pair_body, 0)
        for b in range(2):
            pltpu.make_async_copy(sbufs[b], acc.at[dstm.at[MBLK - 2 + b]],
                                  ssems[b]).wait()
    plsc.subcore_barrier()

    for k in range((NZBLOCKS + NS - 1) // NS):
        b = s + NS * k
        r0 = pl.multiple_of(b * ZROWS, 8)
        if (NS * k) + NS <= NZBLOCKS:
            pltpu.sync_copy(acc.at[pl.ds(r0, ZROWS)],
                            out_hbm.at[c, pl.ds(r0, ZROWS)])
        else:
            @pl.when(b < NZBLOCKS)
            def _():
                pltpu.sync_copy(acc.at[pl.ds(r0, ZROWS)],
                                out_hbm.at[c, pl.ds(r0, ZROWS)])


@jax.jit
def _sc_aggregate(x, srcm, dstm, wm):
    mesh = plsc.VectorSubcoreMesh(core_axis_name="c", subcore_axis_name="s")
    return pl.kernel(
        _sc_body,
        out_type=jax.ShapeDtypeStruct((NC, N_NODES, CH), jnp.float32),
        mesh=mesh,
        scratch_types=[
            pltpu.VMEM((MBLK, CHUNK), jnp.int32),
            pltpu.VMEM((MBLK, CHUNK), jnp.int32),
            pltpu.VMEM((1, WPAD), jnp.float32),
            pltpu.VMEM((CHUNK, CH), jnp.float32),
            pltpu.VMEM((CHUNK, CH), jnp.float32),
            pltpu.VMEM((CHUNK, CH), jnp.float32),
            pltpu.VMEM((CHUNK, CH), jnp.float32),
            pltpu.VMEM_SHARED((N_NODES, CH), jnp.float32),
            pltpu.SemaphoreType.DMA,
            pltpu.SemaphoreType.DMA,
            pltpu.SemaphoreType.DMA,
            pltpu.SemaphoreType.DMA,
        ],
    )(x, srcm, dstm, wm)


def _mm_body(p_ref, w_ref, o_ref):
    s = p_ref[0] + p_ref[1]
    o_ref[...] = jnp.dot(s, w_ref[...], preferred_element_type=jnp.float32)


BLK = 1000


@jax.jit
def _combine_matmul(partials, W):
    return pl.pallas_call(
        _mm_body,
        grid=(N_NODES // BLK,),
        in_specs=[
            pl.BlockSpec((NC, BLK, CH), lambda i: (0, i, 0)),
            pl.BlockSpec((CH, CH), lambda i: (0, 0)),
        ],
        out_specs=pl.BlockSpec((BLK, CH), lambda i: (i, 0)),
        out_shape=jax.ShapeDtypeStruct((N_NODES, CH), jnp.float32),
    )(partials, W)


def kernel(x, W, edge_index, edge_weight):
    src = edge_index[0].astype(jnp.int32).reshape(NW, NMBLK, MBLK, CHUNK)
    dst = edge_index[1].astype(jnp.int32).reshape(NW, NMBLK, MBLK, CHUNK)
    w = jnp.pad(edge_weight.reshape(NW, NMBLK, MBLK * CHUNK),
                ((0, 0), (0, 0), (0, WPAD - MBLK * CHUNK))
                ).reshape(NW, NMBLK, 1, WPAD)
    partials = _sc_aggregate(x, src, dst, w)
    return _combine_matmul(partials, W)
